# baseline (speedup 1.0000x reference)
import functools

import jax
import jax.numpy as jnp
from jax import lax
from jax.experimental import pallas as pl
from jax.experimental.pallas import tpu as pltpu
from jax.experimental.pallas import tpu_sc as plsc

N = 10000
E = 320000
D = 128

NC = 2
NS = 16
NW = NC * NS
EPW = E // NW
C = 40
NCHUNK = EPW // C
NPAD = 10240
NPS = NPAD // NS

_mesh = plsc.VectorSubcoreMesh(
    core_axis_name="c", subcore_axis_name="s", num_cores=NC, num_subcores=NS)


@functools.partial(
    pl.kernel,
    out_type=jax.ShapeDtypeStruct((NC, NPAD, D), jnp.float32),
    mesh=_mesh,
    scratch_types=[
        pltpu.VMEM((8, 2, 1, C), jnp.int32),
        pltpu.VMEM((4, C, D), jnp.float32),
        pltpu.VMEM((2, C, D), jnp.float32),
        pltpu.VMEM((2, C, D), jnp.float32),
        pltpu.VMEM_SHARED((NPAD, D), jnp.float32),
        pltpu.SemaphoreType.DMA((8,)),
        pltpu.SemaphoreType.DMA((4,)),
        pltpu.SemaphoreType.DMA((2,)),
        pltpu.SemaphoreType.DMA((2,)),
    ],
)
def _sc_aggregate(src_hbm, dst_hbm, ea_hbm, x_hbm, part_hbm,
                  ib, xr, eab, sb, agg, isem, gsem, esem, ssem):
    cid = lax.axis_index("c")
    sid = lax.axis_index("s")
    wid = sid * NC + cid

    zeros16 = jnp.zeros((16,), jnp.float32)

    def _zero_row(r, _):
        for j in range(D // 16):
            sb[0, r, pl.ds(j * 16, 16)] = zeros16
        return ()

    lax.fori_loop(0, C, _zero_row, (), unroll=False)
    for k in range(NPS // C):
        pltpu.sync_copy(sb.at[0], agg.at[pl.ds(sid * NPS + k * C, C), :])
    plsc.subcore_barrier()

    def _issue_idx(ci, s):
        pltpu.async_copy(src_hbm.at[wid, ci], ib.at[s, 0], isem.at[s])
        pltpu.async_copy(dst_hbm.at[wid, ci], ib.at[s, 1], isem.at[s])

    def _wait_idx(ci, s):
        pltpu.make_async_copy(src_hbm.at[wid, ci], ib.at[s, 0], isem.at[s]).wait()
        pltpu.make_async_copy(dst_hbm.at[wid, ci], ib.at[s, 1], isem.at[s]).wait()

    def _issue_gather(gx, s):
        pltpu.async_copy(x_hbm.at[ib.at[s, 0, 0]], xr.at[gx], gsem.at[gx])

    def _wait_gather(gx, s):
        pltpu.make_async_copy(
            x_hbm.at[ib.at[s, 0, 0]], xr.at[gx], gsem.at[gx]).wait()

    def _issue_ea(ci, b):
        off = (wid * NCHUNK + ci) * C
        pltpu.async_copy(ea_hbm.at[pl.ds(off, C), :], eab.at[b], esem.at[b])

    def _wait_ea(ci, b):
        off = (wid * NCHUNK + ci) * C
        pltpu.make_async_copy(
            ea_hbm.at[pl.ds(off, C), :], eab.at[b], esem.at[b]).wait()

    def _issue_scatter(b, s):
        pltpu.async_copy(sb.at[b], agg.at[ib.at[s, 1, 0]], ssem.at[b], add=True)

    def _wait_scatter(b, s):
        pltpu.make_async_copy(
            sb.at[b], agg.at[ib.at[s, 1, 0]], ssem.at[b]).wait()

    def _compute(gx, b):
        @plsc.parallel_loop(0, C, 1)
        def _row(r):
            for j in range(D // 16):
                sl = pl.ds(j * 16, 16)
                sb[b, r, sl] = jnp.maximum(xr[gx, r, sl] + eab[b, r, sl], 0.0)

    for s0 in range(4):
        _issue_idx(s0, s0)
    _wait_idx(0, 0)
    _issue_gather(0, 0)
    _wait_idx(1, 1)
    _issue_gather(1, 1)
    _issue_ea(0, 0)
    _issue_ea(1, 1)

    def _group(q, _):
        for k in range(8):
            c = 8 * q + k
            b, gx, s = k % 2, k % 4, k
            s2, s4, gx2 = (k + 2) % 8, (k + 4) % 8, (k + 2) % 4

            @pl.when(c >= 2)
            def _():
                _wait_scatter(b, (k + 6) % 8)

            @pl.when(c + 4 < NCHUNK)
            def _():
                _issue_idx(c + 4, s4)
            _wait_idx(c + 2, s2)
            _issue_gather(gx2, s2)
            _wait_gather(gx, s)
            _wait_ea(c, b)
            _compute(gx, b)
            _issue_ea(c + 2, b)
            _issue_scatter(b, s)
        return ()

    lax.fori_loop(0, (NCHUNK - 2) // 8, _group, (), unroll=False)

    _wait_scatter(0, 6)
    _wait_gather(0, 0)
    _wait_ea(NCHUNK - 2, 0)
    _compute(0, 0)
    _issue_scatter(0, 0)
    _wait_scatter(1, 7)
    _wait_gather(1, 1)
    _wait_ea(NCHUNK - 1, 1)
    _compute(1, 1)
    _issue_scatter(1, 1)
    _wait_scatter(0, 0)
    _wait_scatter(1, 1)

    plsc.subcore_barrier()
    pltpu.sync_copy(agg.at[pl.ds(sid * NPS, NPS), :],
                    part_hbm.at[cid, pl.ds(sid * NPS, NPS), :])


_BN = 1000


def _tc_linear_body(x_ref, p0_ref, p1_ref, w_ref, b_ref, o_ref):
    h = x_ref[...] + p0_ref[0] + p1_ref[0]
    o_ref[...] = lax.dot_general(
        h, w_ref[...], (((1,), (1,)), ((), ())),
        preferred_element_type=jnp.float32) + b_ref[...]


def _tc_linear(x, part, w, b2):
    return pl.pallas_call(
        _tc_linear_body,
        grid=(N // _BN,),
        in_specs=[
            pl.BlockSpec((_BN, D), lambda i: (i, 0)),
            pl.BlockSpec((1, _BN, D), lambda i: (0, i, 0)),
            pl.BlockSpec((1, _BN, D), lambda i: (1, i, 0)),
            pl.BlockSpec((D, D), lambda i: (0, 0)),
            pl.BlockSpec((1, D), lambda i: (0, 0)),
        ],
        out_specs=pl.BlockSpec((_BN, D), lambda i: (i, 0)),
        out_shape=jax.ShapeDtypeStruct((N, D), jnp.float32),
    )(x, part, part, w, b2)


def kernel(x, edge_index, edge_attr, W, b):
    src4 = edge_index[0].reshape(NW, NCHUNK, 1, C)
    dst4 = edge_index[1].reshape(NW, NCHUNK, 1, C)
    part = _sc_aggregate(src4, dst4, edge_attr, x)
    return _tc_linear(x, part, W, b.reshape(1, D))


# speedup vs baseline: 1.0415x; 1.0415x over previous
import functools

import jax
import jax.numpy as jnp
from jax import lax
from jax.experimental import pallas as pl
from jax.experimental.pallas import tpu as pltpu
from jax.experimental.pallas import tpu_sc as plsc

N = 10000
E = 320000
D = 128

NC = 2
NS = 16
NW = NC * NS
EPW = E // NW
C = 40
NCHUNK = EPW // C
NPAD = 10240
NPS = NPAD // NS

_mesh = plsc.VectorSubcoreMesh(
    core_axis_name="c", subcore_axis_name="s", num_cores=NC, num_subcores=NS)


@functools.partial(
    pl.kernel,
    out_type=jax.ShapeDtypeStruct((NC, NPAD, D), jnp.float32),
    mesh=_mesh,
    scratch_types=[
        pltpu.VMEM((8, 2, 1, C), jnp.int32),
        pltpu.VMEM((4, C, D), jnp.float32),
        pltpu.VMEM((2, C, D), jnp.float32),
        pltpu.VMEM((2, C, D), jnp.float32),
        pltpu.VMEM_SHARED((NPAD, D), jnp.float32),
        pltpu.SemaphoreType.DMA((8,)),
        pltpu.SemaphoreType.DMA((4,)),
        pltpu.SemaphoreType.DMA((2,)),
        pltpu.SemaphoreType.DMA((2,)),
    ],
)
def _sc_aggregate(ei_hbm, ea_hbm, x_hbm, part_hbm,
                  ib, xr, eab, sb, agg, isem, gsem, esem, ssem):
    cid = lax.axis_index("c")
    sid = lax.axis_index("s")
    wid = sid * NC + cid

    zeros16 = jnp.zeros((16,), jnp.float32)

    def _zero_row(r, _):
        for j in range(D // 16):
            sb[0, r, pl.ds(j * 16, 16)] = zeros16
        return ()

    lax.fori_loop(0, C, _zero_row, (), unroll=False)
    for k in range(NPS // C):
        pltpu.sync_copy(sb.at[0], agg.at[pl.ds(sid * NPS + k * C, C), :])
    plsc.subcore_barrier()

    def _issue_idx(ci, s):
        pltpu.async_copy(ei_hbm.at[0, wid, ci], ib.at[s, 0], isem.at[s])
        pltpu.async_copy(ei_hbm.at[1, wid, ci], ib.at[s, 1], isem.at[s])

    def _wait_idx(ci, s):
        pltpu.make_async_copy(
            ei_hbm.at[0, wid, ci], ib.at[s, 0], isem.at[s]).wait()
        pltpu.make_async_copy(
            ei_hbm.at[1, wid, ci], ib.at[s, 1], isem.at[s]).wait()

    def _issue_gather(gx, s):
        pltpu.async_copy(x_hbm.at[ib.at[s, 0, 0]], xr.at[gx], gsem.at[gx])

    def _wait_gather(gx, s):
        pltpu.make_async_copy(
            x_hbm.at[ib.at[s, 0, 0]], xr.at[gx], gsem.at[gx]).wait()

    def _issue_ea(ci, b):
        off = (wid * NCHUNK + ci) * C
        pltpu.async_copy(ea_hbm.at[pl.ds(off, C), :], eab.at[b], esem.at[b])

    def _wait_ea(ci, b):
        off = (wid * NCHUNK + ci) * C
        pltpu.make_async_copy(
            ea_hbm.at[pl.ds(off, C), :], eab.at[b], esem.at[b]).wait()

    def _issue_scatter(b, s):
        pltpu.async_copy(sb.at[b], agg.at[ib.at[s, 1, 0]], ssem.at[b], add=True)

    def _wait_scatter(b, s):
        pltpu.make_async_copy(
            sb.at[b], agg.at[ib.at[s, 1, 0]], ssem.at[b]).wait()

    def _compute(gx, b):
        def _row(r, _):
            for j in range(D // 16):
                sl = pl.ds(j * 16, 16)
                sb[b, r, sl] = jnp.maximum(xr[gx, r, sl] + eab[b, r, sl], 0.0)
            return ()

        lax.fori_loop(0, C, _row, (), unroll=False)

    for s0 in range(4):
        _issue_idx(s0, s0)
    _wait_idx(0, 0)
    _issue_gather(0, 0)
    _wait_idx(1, 1)
    _issue_gather(1, 1)
    _issue_ea(0, 0)
    _issue_ea(1, 1)

    def _group(q, _):
        for k in range(8):
            c = 8 * q + k
            b, gx, s = k % 2, k % 4, k
            s2, s4, gx2 = (k + 2) % 8, (k + 4) % 8, (k + 2) % 4

            @pl.when(c >= 2)
            def _():
                _wait_scatter(b, (k + 6) % 8)

            @pl.when(c + 4 < NCHUNK)
            def _():
                _issue_idx(c + 4, s4)
            _wait_idx(c + 2, s2)
            _issue_gather(gx2, s2)
            _wait_gather(gx, s)
            _wait_ea(c, b)
            _compute(gx, b)
            _issue_ea(c + 2, b)
            _issue_scatter(b, s)
        return ()

    lax.fori_loop(0, (NCHUNK - 2) // 8, _group, (), unroll=False)

    _wait_scatter(0, 6)
    _wait_gather(0, 0)
    _wait_ea(NCHUNK - 2, 0)
    _compute(0, 0)
    _issue_scatter(0, 0)
    _wait_scatter(1, 7)
    _wait_gather(1, 1)
    _wait_ea(NCHUNK - 1, 1)
    _compute(1, 1)
    _issue_scatter(1, 1)
    _wait_scatter(0, 0)
    _wait_scatter(1, 1)

    plsc.subcore_barrier()
    pltpu.sync_copy(agg.at[pl.ds(sid * NPS, NPS), :],
                    part_hbm.at[cid, pl.ds(sid * NPS, NPS), :])


_BN = 1000


def _tc_linear_body(x_ref, p0_ref, p1_ref, w_ref, b_ref, o_ref):
    h = x_ref[...] + p0_ref[0] + p1_ref[0]
    o_ref[...] = lax.dot_general(
        h, w_ref[...], (((1,), (1,)), ((), ())),
        preferred_element_type=jnp.float32) + b_ref[...]


def _tc_linear(x, part, w, b2):
    return pl.pallas_call(
        _tc_linear_body,
        grid=(N // _BN,),
        in_specs=[
            pl.BlockSpec((_BN, D), lambda i: (i, 0)),
            pl.BlockSpec((1, _BN, D), lambda i: (0, i, 0)),
            pl.BlockSpec((1, _BN, D), lambda i: (1, i, 0)),
            pl.BlockSpec((D, D), lambda i: (0, 0)),
            pl.BlockSpec((1, D), lambda i: (0, 0)),
        ],
        out_specs=pl.BlockSpec((_BN, D), lambda i: (i, 0)),
        out_shape=jax.ShapeDtypeStruct((N, D), jnp.float32),
    )(x, part, part, w, b2)


def kernel(x, edge_index, edge_attr, W, b):
    ei5 = edge_index.reshape(2, NW, NCHUNK, 1, C)
    part = _sc_aggregate(ei5, edge_attr, x)
    return _tc_linear(x, part, W, b.reshape(1, D))
